# baseline (speedup 1.0000x reference)
import jax
import jax.numpy as jnp
from jax.experimental import pallas as pl
from jax.experimental.pallas import tpu as pltpu

_TILE = 256


def _bf16_round(x):
    return x.astype(jnp.bfloat16).astype(jnp.float32)


def _chamfer_tc_kernel(p_ref, tt_ref, out_ref):
    n = p_ref.shape[0]
    m = tt_ref.shape[1]
    tx = tt_ref[0:1, :]
    ty = tt_ref[1:2, :]
    tz = tt_ref[2:3, :]
    t_sq = tx * tx + ty * ty + tz * tz
    txb = _bf16_round(tx)
    tyb = _bf16_round(ty)
    tzb = _bf16_round(tz)

    def body(i, carry):
        row_sum, cmin = carry
        base = i * _TILE
        px = p_ref[pl.ds(base, _TILE), 0:1]
        py = p_ref[pl.ds(base, _TILE), 1:2]
        pz = p_ref[pl.ds(base, _TILE), 2:3]
        p_sq = px * px + py * py + pz * pz
        dot = _bf16_round(px) * txb
        dot = dot + _bf16_round(py) * tyb
        dot = dot + _bf16_round(pz) * tzb
        d2 = (p_sq + t_sq) - 2.0 * dot
        d2 = jnp.maximum(d2, 0.0)
        rmin = jnp.min(d2, axis=1, keepdims=True)
        row_sum = row_sum + jnp.sum(rmin)
        cmin = jnp.minimum(cmin, jnp.min(d2, axis=0, keepdims=True))
        return row_sum, cmin

    init = (jnp.float32(0.0), jnp.full((1, m), jnp.inf, dtype=jnp.float32))
    row_sum, cmin = jax.lax.fori_loop(0, n // _TILE, body, init)
    loss = row_sum / n + jnp.sum(cmin) / m
    out_ref[:, :] = jnp.reshape(loss, (1, 1))


def kernel(pred, target):
    p = pred[:, :3]
    tt = target[:, :3].T
    out = pl.pallas_call(
        _chamfer_tc_kernel,
        out_shape=jax.ShapeDtypeStruct((1, 1), jnp.float32),
    )(p, tt)
    return out[0, 0]
